# baseline (speedup 1.0000x reference)
import functools

import jax
import jax.numpy as jnp
from jax import lax
from jax.experimental import pallas as pl
from jax.experimental.pallas import tpu as pltpu
from jax.experimental.pallas import tpu_sc as plsc

_LF = 0.15875
_VGOAL = 0.9
_N = 100000
_NS = 16
_L = 16
_RPW = _N // _NS
_TWIN = 6400
_H = _TWIN // 2
_BASE_MAX = 93568
_UNROLL = 10
_TAIL0 = 99968
_TAILN = _N - _TAIL0

_BIG = 3.4e38


def _atan(u):
  t = jnp.abs(u)
  inv = t > 1.0
  z = jnp.where(inv, 1.0 / jnp.maximum(t, 1e-30), t)
  big = z > 0.4142135623730951
  z2 = jnp.where(big, (z - 1.0) / (z + 1.0), z)
  w = z2 * z2
  p = ((8.05374449538e-2 * w - 1.38776856032e-1) * w
       + 1.99777106478e-1) * w - 3.33329491539e-1
  r = z2 + z2 * w * p
  r = jnp.where(big, jnp.float32(0.7853981633974483) + r, r)
  r = jnp.where(inv, jnp.float32(1.5707963267948966) - r, r)
  return jnp.where(u < 0.0, -r, r)


def _sincos(t):
  tt = t * jnp.float32(0.6366197723675814)
  n = (tt + jnp.float32(0.5) * jnp.sign(tt)).astype(jnp.int32)
  nf = n.astype(jnp.float32)
  r = (t - nf * jnp.float32(1.5707962513e+00)) - nf * jnp.float32(7.5497894159e-08)
  q = n & 3
  r2 = r * r
  sp = r + r * r2 * (jnp.float32(-1.6666654611e-01)
                     + r2 * (jnp.float32(8.3321608736e-03)
                             + r2 * jnp.float32(-1.9515295891e-04)))
  cp = 1.0 + r2 * (jnp.float32(-0.5)
                   + r2 * (jnp.float32(4.166664568298827e-02)
                           + r2 * (jnp.float32(-1.388731625493765e-03)
                                   + r2 * jnp.float32(2.443315711809948e-05))))
  s = jnp.where(q == 0, sp, jnp.where(q == 1, cp,
                                      jnp.where(q == 2, -sp, -cp)))
  c = jnp.where(q == 0, cp, jnp.where(q == 1, -sp,
                                      jnp.where(q == 2, -cp, sp)))
  return s, c


def _sc_body(wt_hbm, pk_hbm, tail_hbm, out_hbm,
             buf_v, pv_v, tail_v, rec_v, outs_v, shared_v, semb, semb2):
  sid = lax.axis_index("s")
  row0 = sid * _RPW
  base = pl.multiple_of(jnp.minimum((row0 // 128) * 128, _BASE_MAX), 128)
  cb = pltpu.async_copy(wt_hbm.at[:, pl.ds(base, _H)],
                        buf_v.at[:, pl.ds(0, _H)], semb)
  cb2 = pltpu.async_copy(wt_hbm.at[:, pl.ds(base + _H, _H)],
                         buf_v.at[:, pl.ds(_H, _H)], semb2)
  pltpu.sync_copy(pk_hbm, pv_v.at[pl.ds(0, 8)])
  pltpu.sync_copy(tail_hbm, tail_v)

  lanes = lax.broadcasted_iota(jnp.int32, (_L,), 0)
  zeros_i = jnp.zeros((_L,), jnp.int32)
  pv16 = pv_v[...]
  pcx = jnp.sum(jnp.where(lanes == 0, pv16, 0.0))
  pcy = jnp.sum(jnp.where(lanes == 1, pv16, 0.0))

  def step(i, bd, br):
    xv = buf_v[1, pl.ds(i, _L)]
    yv = buf_v[2, pl.ds(i, _L)]
    dx = xv - pcx
    dy = yv - pcy
    d2 = dx * dx + dy * dy
    upd = d2 < bd
    return jnp.where(upd, d2, bd), jnp.where(upd, i + lanes, br)

  bd0 = jnp.full((_L,), _BIG, jnp.float32)
  cb.wait()

  @plsc.parallel_loop(0, _H, _L, unroll=_UNROLL, carry=(bd0, zeros_i))
  def _loop(i, carry):
    bd, br = carry
    return step(i, bd, br)

  bd, br = _loop
  cb2.wait()

  @plsc.parallel_loop(_H, _TWIN, _L, unroll=_UNROLL, carry=(bd, br))
  def _loop2(i, carry):
    bd, br = carry
    return step(i, bd, br)

  bd, br = _loop2

  m = jnp.min(bd)
  c_win = jnp.min(jnp.where(bd == m, br, jnp.int32(0x7FFFFFFF)))
  g_row = (base + c_win).astype(jnp.float32)
  co = pl.multiple_of((c_win // _L) * _L, _L)
  jsel = lanes == (c_win - co)
  xw = jnp.sum(jnp.where(jsel, buf_v[1, pl.ds(co, _L)], 0.0))
  yw = jnp.sum(jnp.where(jsel, buf_v[2, pl.ds(co, _L)], 0.0))
  hw = jnp.sum(jnp.where(jsel, buf_v[3, pl.ds(co, _L)], 0.0))
  sw = jnp.sum(jnp.where(jsel, buf_v[5, pl.ds(co, _L)], 0.0))
  outvec = jnp.where(lanes == 0, m,
                     jnp.where(lanes == 1, g_row,
                               jnp.where(lanes == 3, xw,
                                         jnp.where(lanes == 4, yw,
                                                   jnp.where(lanes == 5, hw,
                                                             sw)))))
  outs_v[...] = outvec
  pltpu.sync_copy(outs_v, shared_v.at[pl.ds(sid * _L, _L)])
  plsc.subcore_barrier()

  @pl.when(sid == 0)
  def _final():
    pltpu.sync_copy(shared_v, rec_v)
    d2v = plsc.load_gather(rec_v, [lanes * _L])
    grv = plsc.load_gather(rec_v, [lanes * _L + 1])
    mm = jnp.min(d2v)
    keyv = grv * jnp.float32(_NS) + lanes.astype(jnp.float32)
    kstar = jnp.min(jnp.where(d2v <= mm, keyv, _BIG))
    wsel = keyv == kstar
    xs = jnp.sum(jnp.where(wsel, plsc.load_gather(rec_v, [lanes * _L + 3]), 0.0))
    ys = jnp.sum(jnp.where(wsel, plsc.load_gather(rec_v, [lanes * _L + 4]), 0.0))
    hs = jnp.sum(jnp.where(wsel, plsc.load_gather(rec_v, [lanes * _L + 5]), 0.0))
    ss = jnp.sum(jnp.where(wsel, plsc.load_gather(rec_v, [lanes * _L + 7]), 0.0))
    tbd = jnp.full((_L,), _BIG, jnp.float32)
    tbr = zeros_i
    for k in (0, _L):
      xv = tail_v[pl.ds(k, _L)]
      yv = tail_v[pl.ds(32 + k, _L)]
      dx = xv - pcx
      dy = yv - pcy
      d2 = dx * dx + dy * dy
      upd = d2 < tbd
      tbd = jnp.where(upd, d2, tbd)
      tbr = jnp.where(upd, k + lanes, tbr)
    d2t = jnp.min(tbd)
    ct = jnp.min(jnp.where(tbd == d2t, tbr, jnp.int32(0x7FFFFFFF)))
    cto = pl.multiple_of((ct // _L) * _L, _L)
    tsel = lanes == (ct - cto)
    xt = jnp.sum(jnp.where(tsel, tail_v[pl.ds(cto, _L)], 0.0))
    yt = jnp.sum(jnp.where(tsel, tail_v[pl.ds(32 + cto, _L)], 0.0))
    ht = jnp.sum(jnp.where(tsel, tail_v[pl.ds(64 + cto, _L)], 0.0))
    st = jnp.sum(jnp.where(tsel, tail_v[pl.ds(96 + cto, _L)], 0.0))
    zf = jnp.zeros((_L,), jnp.float32)
    use_t = (d2t + zf) < (mm + zf)
    wx = jnp.where(use_t, xt + zf, xs + zf)
    wy = jnp.where(use_t, yt + zf, ys + zf)
    wh = jnp.where(use_t, ht + zf, hs + zf)
    ws = jnp.where(use_t, st + zf, ss + zf)

    k_e = jnp.sum(jnp.where(lanes == 2, pv16, 0.0)) + zf
    k_h = jnp.sum(jnp.where(lanes == 3, pv16, 0.0)) + zf
    theta = jnp.sum(jnp.where(lanes == 4, pv16, 0.0)) + zf
    pi = jnp.float32(jnp.pi)
    thetap = jnp.remainder(theta + pi, 2.0 * pi)
    s, c = _sincos(thetap + pi / 2.0)
    fav0 = -c
    fav1 = -s
    ce = ((pcx + zf) - wx) * fav0 + ((pcy + zf) - wy) * fav1
    he = jnp.remainder(wh - thetap + pi, 2.0 * pi) - pi
    v = ws * jnp.float32(_VGOAL)
    steer = k_h * he + _atan(k_e * -ce / (v + 1e-05))
    res = jnp.where(lanes == 0, steer,
                    jnp.where(lanes == 1, v,
                              jnp.where(lanes == 2, ce,
                                        jnp.where(lanes == 3, he, zf))))
    outs_v[...] = res
    pltpu.sync_copy(outs_v, out_hbm.at[0])


@functools.cache
def _get_sc():
  return pl.kernel(
      _sc_body,
      out_type=jax.ShapeDtypeStruct((1, _L), jnp.float32),
      mesh=plsc.VectorSubcoreMesh(core_axis_name="c", subcore_axis_name="s",
                                  num_cores=1, num_subcores=_NS),
      compiler_params=pltpu.CompilerParams(needs_layout_passes=False,
                                           use_tc_tiling_on_sc=True),
      scratch_types=[
          pltpu.VMEM((6, _TWIN), jnp.float32),
          pltpu.VMEM((_L,), jnp.float32),
          pltpu.VMEM((128,), jnp.float32),
          pltpu.VMEM((_NS * _L,), jnp.float32),
          pltpu.VMEM((_L,), jnp.float32),
          pltpu.VMEM_SHARED((_NS * _L,), jnp.float32),
          pltpu.SemaphoreType.DMA,
          pltpu.SemaphoreType.DMA,
      ],
  )


def kernel(pose, waypoints, k_e, k_h):
  s2 = jnp.sin(pose[2])
  c2 = jnp.cos(pose[2])
  pk = jnp.stack([pose[0] + _LF * s2, pose[1] + _LF * c2,
                  k_e.astype(jnp.float32), k_h.astype(jnp.float32),
                  pose[2], jnp.float32(0.0), jnp.float32(0.0),
                  jnp.float32(0.0)])
  tail = jnp.concatenate([waypoints[_TAIL0:, 1], waypoints[_TAIL0:, 2],
                          waypoints[_TAIL0:, 3], waypoints[_TAIL0:, 5]])
  wt = waypoints.T
  out = _get_sc()(wt, pk, tail)
  return (out[0, 0], out[0, 1], out[0, 2], out[0, 3])
